# baseline (speedup 1.0000x reference)
import functools

import jax
import jax.numpy as jnp
from jax import lax
from jax.experimental import pallas as pl
from jax.experimental.pallas import tpu as pltpu
from jax.experimental.pallas import tpu_sc as plsc

_B = 4096
_D = 256
_K = 1024
_L = 4
_BB = 256
_NW = 32
_RPW = _B // _NW

_CONTRACT_LAST = (((1,), (1,)), ((), ()))


def _rows_body(scal_ref, cb_ref, out_ref):
    inv_t = scal_ref[0, 1]
    cb = cb_ref[...]
    cbn = jnp.sum(cb * cb, axis=-1)
    out_ref[...] = jnp.broadcast_to(cbn[None], (8, 3, _K)) * (-inv_t)


def _sc_fill(row8_hbm, out_hbm, row8_v, rep_sh):
    s = lax.axis_index("s")
    c = lax.axis_index("c")
    wid = s * 2 + c
    pltpu.sync_copy(row8_hbm, row8_v)
    pltpu.sync_copy(row8_v, rep_sh.at[pl.ds(s * 8, 8), :])
    plsc.subcore_barrier()
    pltpu.sync_copy(rep_sh,
                    out_hbm.at[pl.ds(wid * _RPW, _RPW), pl.ds(_K, 3 * _K)])


def _main_body(sc_ref, scal_ref, feat_ref, wh_ref, wl_ref, bias_ref, g_ref,
               beta_ref, cb0_ref, cb0h_ref, logits_ref, qsum_ref, nrow_ref):
    del sc_ref
    s0 = scal_ref[0, 0]
    inv_t = scal_ref[0, 1]

    @pl.when(pl.program_id(0) == 0)
    def _():
        cb0 = cb0_ref[...]
        nrow_ref[0, :] = jnp.sum(cb0 * cb0, axis=-1) * (-inv_t)

    f = feat_ref[...]
    fh = f.astype(jnp.bfloat16)
    fl = (f - fh.astype(jnp.float32)).astype(jnp.bfloat16)
    h = (jax.lax.dot_general(fh, wh_ref[...], _CONTRACT_LAST,
                             preferred_element_type=jnp.float32)
         + jax.lax.dot_general(fh, wl_ref[...], _CONTRACT_LAST,
                               preferred_element_type=jnp.float32)
         + jax.lax.dot_general(fl, wh_ref[...], _CONTRACT_LAST,
                               preferred_element_type=jnp.float32))
    h = h + bias_ref[...]

    mu = jnp.mean(h, axis=-1, keepdims=True)
    var = jnp.mean((h - mu) * (h - mu), axis=-1, keepdims=True)
    h = (h - mu) * jax.lax.rsqrt(var + 1e-5)
    h = h * g_ref[...] + beta_ref[...]
    h = jnp.maximum(h, 0.0)

    rs = h * s0

    rown = jnp.sum(rs * rs, axis=-1, keepdims=True)
    cross = jax.lax.dot_general(rs.astype(jnp.bfloat16), cb0h_ref[...],
                                _CONTRACT_LAST,
                                preferred_element_type=jnp.float32)
    logits_ref[...] = ((2.0 * inv_t) * cross - inv_t * rown
                       + nrow_ref[0][None, :])

    mu2 = jnp.mean(rs, axis=-1, keepdims=True)
    var2 = jnp.mean((rs - mu2) * (rs - mu2), axis=-1, keepdims=True)
    qsum_ref[...] = (rs - mu2) * jax.lax.rsqrt(var2 + 1e-5)


def kernel(features, W_proj, b_proj, ln_gamma, ln_beta, codebooks,
           residual_scales, temperature):
    inv_t = 1.0 / jnp.maximum(temperature, 0.01)
    scal = jnp.stack([residual_scales[0].astype(jnp.float32),
                      inv_t.astype(jnp.float32)]).reshape(1, 2)

    w_hi = W_proj.astype(jnp.bfloat16)
    w_lo = (W_proj - w_hi.astype(jnp.float32)).astype(jnp.bfloat16)
    cb0 = codebooks[0]
    cb0_hi = cb0.astype(jnp.bfloat16)

    rows8 = pl.pallas_call(
        _rows_body,
        in_specs=[
            pl.BlockSpec(memory_space=pltpu.SMEM),
            pl.BlockSpec((_L - 1, _K, _D), lambda: (0, 0, 0)),
        ],
        out_specs=pl.BlockSpec((8, _L - 1, _K), lambda: (0, 0, 0)),
        out_shape=jax.ShapeDtypeStruct((8, _L - 1, _K), jnp.float32),
    )(scal, codebooks[1:]).reshape(8, (_L - 1) * _K)

    mesh = plsc.VectorSubcoreMesh(core_axis_name="c", subcore_axis_name="s")
    sc_fill = functools.partial(
        pl.kernel,
        out_type=jax.ShapeDtypeStruct((_B, _L * _K), jnp.float32),
        mesh=mesh,
        scratch_types=[
            pltpu.VMEM((8, (_L - 1) * _K), jnp.float32),
            pltpu.VMEM_SHARED((_RPW, (_L - 1) * _K), jnp.float32),
        ],
    )(_sc_fill)
    sc_buf = sc_fill(rows8)

    grid = (_B // _BB,)
    logits2d, qsum = pl.pallas_call(
        _main_body,
        grid=grid,
        in_specs=[
            pl.BlockSpec(memory_space=pl.ANY),
            pl.BlockSpec(memory_space=pltpu.SMEM),
            pl.BlockSpec((_BB, _D), lambda i: (i, 0)),
            pl.BlockSpec((_D, _D), lambda i: (0, 0)),
            pl.BlockSpec((_D, _D), lambda i: (0, 0)),
            pl.BlockSpec((1, _D), lambda i: (0, 0)),
            pl.BlockSpec((1, _D), lambda i: (0, 0)),
            pl.BlockSpec((1, _D), lambda i: (0, 0)),
            pl.BlockSpec((_K, _D), lambda i: (0, 0)),
            pl.BlockSpec((_K, _D), lambda i: (0, 0)),
        ],
        out_specs=[
            pl.BlockSpec((_BB, _K), lambda i: (i, 0)),
            pl.BlockSpec((_BB, _D), lambda i: (i, 0)),
        ],
        out_shape=[
            jax.ShapeDtypeStruct((_B, _L * _K), jnp.float32),
            jax.ShapeDtypeStruct((_B, _D), jnp.float32),
        ],
        input_output_aliases={0: 0},
        scratch_shapes=[pltpu.VMEM((1, _K), jnp.float32)],
        compiler_params=pltpu.CompilerParams(
            dimension_semantics=("arbitrary",)),
    )(
        sc_buf,
        scal,
        features,
        w_hi,
        w_lo,
        b_proj.reshape(1, _D),
        ln_gamma.reshape(1, _D),
        ln_beta.reshape(1, _D),
        cb0,
        cb0_hi,
    )
    return logits2d.reshape(_B, _L, _K), qsum


# speedup vs baseline: 1.4836x; 1.4836x over previous
import jax
import jax.numpy as jnp
from jax.experimental import pallas as pl
from jax.experimental.pallas import tpu as pltpu

_B = 4096
_D = 256
_K = 1024
_L = 4
_BB = 512

_CONTRACT_LAST = (((1,), (1,)), ((), ()))


def _body(scal_ref, feat_ref, w_ref, bias_ref, g_ref, beta_ref, cb_ref,
          logits_ref, qsum_ref, nrow_ref):
    s0 = scal_ref[0, 0]
    inv_t = scal_ref[0, 1]

    @pl.when(pl.program_id(0) == 0)
    def _():
        cb = cb_ref[...]
        nrow_ref[...] = jnp.sum(cb * cb, axis=-1) * (-inv_t)

    f = feat_ref[...]
    w = w_ref[...]
    h = jax.lax.dot_general(f, w, _CONTRACT_LAST,
                            preferred_element_type=jnp.float32)
    h = h + bias_ref[...]

    mu = jnp.mean(h, axis=-1, keepdims=True)
    var = jnp.mean((h - mu) * (h - mu), axis=-1, keepdims=True)
    h = (h - mu) * jax.lax.rsqrt(var + 1e-5)
    h = h * g_ref[...] + beta_ref[...]
    h = jnp.maximum(h, 0.0)

    rs = h * s0

    rown = jnp.sum(rs * rs, axis=-1, keepdims=True)
    cb0 = cb_ref[0]
    cross = jax.lax.dot_general(rs, cb0, _CONTRACT_LAST,
                                preferred_element_type=jnp.float32)
    logits_ref[:, 0:_K] = ((2.0 * inv_t) * cross - inv_t * rown
                           + nrow_ref[0][None, :])

    for lvl in range(1, _L):
        logits_ref[:, lvl * _K:(lvl + 1) * _K] = jnp.broadcast_to(
            nrow_ref[lvl][None, :], (_BB, _K))

    mu2 = jnp.mean(rs, axis=-1, keepdims=True)
    var2 = jnp.mean((rs - mu2) * (rs - mu2), axis=-1, keepdims=True)
    qsum_ref[...] = (rs - mu2) * jax.lax.rsqrt(var2 + 1e-5)


def kernel(features, W_proj, b_proj, ln_gamma, ln_beta, codebooks,
           residual_scales, temperature):
    inv_t = 1.0 / jnp.maximum(temperature, 0.01)
    scal = jnp.stack([residual_scales[0].astype(jnp.float32),
                      inv_t.astype(jnp.float32)]).reshape(1, 2)

    grid = (_B // _BB,)
    logits2d, qsum = pl.pallas_call(
        _body,
        grid=grid,
        in_specs=[
            pl.BlockSpec(memory_space=pltpu.SMEM),
            pl.BlockSpec((_BB, _D), lambda i: (i, 0)),
            pl.BlockSpec((_D, _D), lambda i: (0, 0)),
            pl.BlockSpec((1, _D), lambda i: (0, 0)),
            pl.BlockSpec((1, _D), lambda i: (0, 0)),
            pl.BlockSpec((1, _D), lambda i: (0, 0)),
            pl.BlockSpec((_L, _K, _D), lambda i: (0, 0, 0)),
        ],
        out_specs=[
            pl.BlockSpec((_BB, _L * _K), lambda i: (i, 0)),
            pl.BlockSpec((_BB, _D), lambda i: (i, 0)),
        ],
        out_shape=[
            jax.ShapeDtypeStruct((_B, _L * _K), jnp.float32),
            jax.ShapeDtypeStruct((_B, _D), jnp.float32),
        ],
        scratch_shapes=[pltpu.VMEM((_L, _K), jnp.float32)],
        compiler_params=pltpu.CompilerParams(
            dimension_semantics=("arbitrary",)),
    )(
        scal,
        features,
        W_proj,
        b_proj.reshape(1, _D),
        ln_gamma.reshape(1, _D),
        ln_beta.reshape(1, _D),
        codebooks,
    )
    return logits2d.reshape(_B, _L, _K), qsum
